# baseline (speedup 1.0000x reference)
import functools

import jax
import jax.numpy as jnp
from jax import lax
from jax.experimental import pallas as pl
from jax.experimental.pallas import tpu as pltpu
from jax.experimental.pallas import tpu_sc as plsc

N = 10000
D = 128
NG = 128
R = 1000
NB = N // R
NP = 10240
CH = 48
NW = 32
RPT = NP // 16


@functools.lru_cache(maxsize=None)
def _make_scatter(EP):
    chunks = EP // (NW * CH)
    epw = EP // NW
    mesh = plsc.VectorSubcoreMesh(core_axis_name="c", subcore_axis_name="s")

    @functools.partial(
        pl.kernel, mesh=mesh,
        out_type=jax.ShapeDtypeStruct((2, NP, D), jnp.float32),
        scratch_types=[
            pltpu.VMEM((CH,), jnp.int32),
            pltpu.VMEM((CH,), jnp.int32),
            pltpu.VMEM((CH, D), jnp.float32),
            pltpu.VMEM_SHARED((NP, D), jnp.float32),
            pltpu.SemaphoreType.DMA,
        ])
    def k(src_hbm, dst_hbm, h_hbm, zeros_hbm, out_hbm, srcv, dstv, rows, acc, sem):
        c = lax.axis_index("c")
        s = lax.axis_index("s")
        w = c * 16 + s
        pltpu.sync_copy(zeros_hbm.at[pl.ds(s * RPT, RPT)],
                        acc.at[pl.ds(s * RPT, RPT)])
        plsc.subcore_barrier()

        def body(kk, carry):
            off = w * epw + kk * CH
            pltpu.sync_copy(src_hbm.at[pl.ds(off, CH)], srcv)
            pltpu.sync_copy(dst_hbm.at[pl.ds(off, CH)], dstv)
            pltpu.async_copy(h_hbm.at[srcv], rows, sem).wait()
            pltpu.sync_copy(rows, acc.at[dstv], add=True)
            return carry

        lax.fori_loop(0, chunks, body, 0)
        plsc.subcore_barrier()
        pltpu.sync_copy(acc.at[pl.ds(s * RPT, RPT)],
                        out_hbm.at[c].at[pl.ds(s * RPT, RPT)])

    return k


def _layer_tc(h, agg, W1, b1, W2, b2, gamma, beta, batch3):
    def body(h_ref, agg_ref, w1_ref, b1_ref, w2_ref, b2_ref, g_ref, bt_ref,
             bat_ref, hbn_ref, pooled_ref, v_all, stats):
        p = pl.program_id(0)
        i = pl.program_id(1)

        @pl.when(p == 0)
        def _p0():
            u = h_ref[...] + agg_ref[0] + agg_ref[1]
            t = jnp.maximum(
                jnp.dot(u, w1_ref[...], preferred_element_type=jnp.float32)
                + b1_ref[...], 0.0)
            v = jnp.maximum(
                jnp.dot(t, w2_ref[...], preferred_element_type=jnp.float32)
                + b2_ref[...], 0.0)
            v_all[pl.ds(i * R, R), :] = v
            hbn_ref[...] = v

            @pl.when(i == 0)
            def _():
                stats[...] = jnp.zeros_like(stats)

            stats[0:1, :] += jnp.sum(v, axis=0, keepdims=True)
            stats[1:2, :] += jnp.sum(v * v, axis=0, keepdims=True)

        @pl.when(p == 1)
        def _p1():
            @pl.when(i == 0)
            def _():
                mu = stats[0:1, :] * (1.0 / N)
                var = stats[1:2, :] * (1.0 / N) - mu * mu
                a = g_ref[...] * lax.rsqrt(var + 1e-5)
                stats[2:3, :] = a
                stats[3:4, :] = bt_ref[...] - a * mu

            a = stats[2:3, :]
            cc = stats[3:4, :]
            v = v_all[pl.ds(i * R, R), :]
            hb = a * v + cc
            hbn_ref[...] = hb
            bblk = bat_ref[...].reshape(1, R)
            oh = (lax.broadcasted_iota(jnp.int32, (NG, R), 0)
                  == bblk).astype(jnp.float32)
            contrib = jnp.dot(oh, hb, preferred_element_type=jnp.float32)

            @pl.when(i == 0)
            def _():
                pooled_ref[...] = contrib

            @pl.when(i != 0)
            def _():
                pooled_ref[...] += contrib

    return pl.pallas_call(
        body,
        grid=(2, NB),
        in_specs=[
            pl.BlockSpec((R, D), lambda p, i: (i, 0)),
            pl.BlockSpec((2, R, D), lambda p, i: (0, i, 0)),
            pl.BlockSpec((D, D), lambda p, i: (0, 0)),
            pl.BlockSpec((1, D), lambda p, i: (0, 0)),
            pl.BlockSpec((D, D), lambda p, i: (0, 0)),
            pl.BlockSpec((1, D), lambda p, i: (0, 0)),
            pl.BlockSpec((1, D), lambda p, i: (0, 0)),
            pl.BlockSpec((1, D), lambda p, i: (0, 0)),
            pl.BlockSpec((1, 1, R), lambda p, i: (i, 0, 0)),
        ],
        out_specs=[
            pl.BlockSpec((R, D), lambda p, i: (i, 0)),
            pl.BlockSpec((NG, D), lambda p, i: (0, 0)),
        ],
        out_shape=[
            jax.ShapeDtypeStruct((N, D), jnp.float32),
            jax.ShapeDtypeStruct((NG, D), jnp.float32),
        ],
        scratch_shapes=[
            pltpu.VMEM((N, D), jnp.float32),
            pltpu.VMEM((8, D), jnp.float32),
        ],
    )(h, agg, W1, b1, W2, b2, gamma, beta, batch3)


def _pad_edges(ei):
    E = ei.shape[1]
    EP = -(-E // (NW * CH)) * (NW * CH)
    pad = EP - E
    src = jnp.concatenate([ei[0], jnp.zeros((pad,), jnp.int32)])
    dst = jnp.concatenate([ei[1], jnp.full((pad,), N, jnp.int32)])
    return src, dst, EP


def kernel(x, aug_x, edge_index, aug_edge_index, id_mat, batch, params):
    batch3 = batch.reshape(NB, 1, R)
    zeros = jnp.zeros((NP, D), jnp.float32)
    p2 = [{k: (v.reshape(1, D) if v.ndim == 1 else v) for k, v in p.items()}
          for p in params]

    def branch(x0, ei):
        src, dst, EP = _pad_edges(ei)
        scat = _make_scatter(EP)
        h = x0
        outs = []
        for p in p2:
            agg = scat(src, dst, h, zeros)
            h, pooled = _layer_tc(h, agg, p['W1'], p['b1'], p['W2'], p['b2'],
                                  p['gamma'], p['beta'], batch3)
            outs.append(pooled)
        return jnp.concatenate(outs, axis=1)

    con1 = branch(x, edge_index)
    con2 = branch(x, aug_edge_index)
    sem1 = branch(x, id_mat)
    sem2 = branch(aug_x, id_mat)
    return (con1, con2, sem1, sem2)


# speedup vs baseline: 1.1480x; 1.1480x over previous
import functools

import jax
import jax.numpy as jnp
from jax import lax
from jax.experimental import pallas as pl
from jax.experimental.pallas import tpu as pltpu
from jax.experimental.pallas import tpu_sc as plsc

N = 10000
D = 128
NG = 128
R = 1000
NB = N // R
NP = 10240
CH = 64
NW = 32
RPT = NP // 16


@functools.lru_cache(maxsize=None)
def _make_scatter(EP):
    chunks = EP // (NW * CH)
    epw = EP // NW
    mesh = plsc.VectorSubcoreMesh(core_axis_name="c", subcore_axis_name="s")

    @functools.partial(
        pl.kernel, mesh=mesh,
        out_type=jax.ShapeDtypeStruct((2, NP, D), jnp.float32),
        scratch_types=[
            pltpu.VMEM((CH,), jnp.int32),
            pltpu.VMEM((CH,), jnp.int32),
            pltpu.VMEM((CH, D), jnp.float32),
            pltpu.VMEM_SHARED((NP, D), jnp.float32),
            pltpu.SemaphoreType.DMA,
        ])
    def k(src_hbm, dst_hbm, h_hbm, zeros_hbm, out_hbm, srcv, dstv, rows, acc, sem):
        c = lax.axis_index("c")
        s = lax.axis_index("s")
        w = c * 16 + s
        pltpu.sync_copy(zeros_hbm.at[pl.ds(s * RPT, RPT)],
                        acc.at[pl.ds(s * RPT, RPT)])
        plsc.subcore_barrier()

        def body(kk, carry):
            off = w * epw + kk * CH
            pltpu.sync_copy(src_hbm.at[pl.ds(off, CH)], srcv)
            pltpu.sync_copy(dst_hbm.at[pl.ds(off, CH)], dstv)
            pltpu.async_copy(h_hbm.at[srcv], rows, sem).wait()
            pltpu.sync_copy(rows, acc.at[dstv], add=True)
            return carry

        lax.fori_loop(0, chunks, body, 0)
        plsc.subcore_barrier()
        pltpu.sync_copy(acc.at[pl.ds(s * RPT, RPT)],
                        out_hbm.at[c].at[pl.ds(s * RPT, RPT)])

    return k


def _layer_tc(h, agg, W1, b1, W2, b2, gamma, beta, batch3):
    def body(h_ref, agg_ref, w1_ref, b1_ref, w2_ref, b2_ref, g_ref, bt_ref,
             bat_ref, hbn_ref, pooled_ref, v_all, stats):
        p = pl.program_id(0)
        i = pl.program_id(1)

        @pl.when(p == 0)
        def _p0():
            u = h_ref[...] + agg_ref[0] + agg_ref[1]
            t = jnp.maximum(
                jnp.dot(u, w1_ref[...], preferred_element_type=jnp.float32)
                + b1_ref[...], 0.0)
            v = jnp.maximum(
                jnp.dot(t, w2_ref[...], preferred_element_type=jnp.float32)
                + b2_ref[...], 0.0)
            v_all[pl.ds(i * R, R), :] = v
            hbn_ref[...] = v

            @pl.when(i == 0)
            def _():
                stats[...] = jnp.zeros_like(stats)

            stats[0:1, :] += jnp.sum(v, axis=0, keepdims=True)
            stats[1:2, :] += jnp.sum(v * v, axis=0, keepdims=True)

        @pl.when(p == 1)
        def _p1():
            @pl.when(i == 0)
            def _():
                mu = stats[0:1, :] * (1.0 / N)
                var = stats[1:2, :] * (1.0 / N) - mu * mu
                a = g_ref[...] * lax.rsqrt(var + 1e-5)
                stats[2:3, :] = a
                stats[3:4, :] = bt_ref[...] - a * mu

            a = stats[2:3, :]
            cc = stats[3:4, :]
            v = v_all[pl.ds(i * R, R), :]
            hb = a * v + cc
            hbn_ref[...] = hb
            bblk = bat_ref[...].reshape(1, R)
            oh = (lax.broadcasted_iota(jnp.int32, (NG, R), 0)
                  == bblk).astype(jnp.float32)
            contrib = jnp.dot(oh, hb, preferred_element_type=jnp.float32)

            @pl.when(i == 0)
            def _():
                pooled_ref[...] = contrib

            @pl.when(i != 0)
            def _():
                pooled_ref[...] += contrib

    return pl.pallas_call(
        body,
        grid=(2, NB),
        in_specs=[
            pl.BlockSpec((R, D), lambda p, i: (i, 0)),
            pl.BlockSpec((2, R, D), lambda p, i: (0, i, 0)),
            pl.BlockSpec((D, D), lambda p, i: (0, 0)),
            pl.BlockSpec((1, D), lambda p, i: (0, 0)),
            pl.BlockSpec((D, D), lambda p, i: (0, 0)),
            pl.BlockSpec((1, D), lambda p, i: (0, 0)),
            pl.BlockSpec((1, D), lambda p, i: (0, 0)),
            pl.BlockSpec((1, D), lambda p, i: (0, 0)),
            pl.BlockSpec((1, 1, R), lambda p, i: (i, 0, 0)),
        ],
        out_specs=[
            pl.BlockSpec((R, D), lambda p, i: (i, 0)),
            pl.BlockSpec((NG, D), lambda p, i: (0, 0)),
        ],
        out_shape=[
            jax.ShapeDtypeStruct((N, D), jnp.float32),
            jax.ShapeDtypeStruct((NG, D), jnp.float32),
        ],
        scratch_shapes=[
            pltpu.VMEM((N, D), jnp.float32),
            pltpu.VMEM((8, D), jnp.float32),
        ],
    )(h, agg, W1, b1, W2, b2, gamma, beta, batch3)


def _pad_edges(ei):
    E = ei.shape[1]
    EP = -(-E // (NW * CH)) * (NW * CH)
    pad = EP - E
    src = jnp.concatenate([ei[0], jnp.zeros((pad,), jnp.int32)])
    dst = jnp.concatenate([ei[1], jnp.full((pad,), N, jnp.int32)])
    return src, dst, EP


def kernel(x, aug_x, edge_index, aug_edge_index, id_mat, batch, params):
    batch3 = batch.reshape(NB, 1, R)
    zeros = jnp.zeros((NP, D), jnp.float32)
    p2 = [{k: (v.reshape(1, D) if v.ndim == 1 else v) for k, v in p.items()}
          for p in params]

    def branch(x0, ei):
        src, dst, EP = _pad_edges(ei)
        scat = _make_scatter(EP)
        h = x0
        outs = []
        for p in p2:
            agg = scat(src, dst, h, zeros)
            h, pooled = _layer_tc(h, agg, p['W1'], p['b1'], p['W2'], p['b2'],
                                  p['gamma'], p['beta'], batch3)
            outs.append(pooled)
        return jnp.concatenate(outs, axis=1)

    con1 = branch(x, edge_index)
    con2 = branch(x, aug_edge_index)
    sem1 = branch(x, id_mat)
    sem2 = branch(aug_x, id_mat)
    return (con1, con2, sem1, sem2)
